# Initial kernel scaffold; baseline (speedup 1.0000x reference)
import jax
import jax.numpy as jnp
from jax.experimental import pallas as pl


def kernel(x, ln1_g, ln1_b, ln2_g, ln2_b, ln3_g, ln3_b, w_qkv, b_qkv, w_o, b_o, w_gate, b_gate, w_e1, b_e1, w_e2, b_e2, w_m1, b_m1, w_m2, b_m2):
    raise NotImplementedError("write your pallas kernel here")


# speedup vs baseline: 2.1579x; 2.1579x over previous
import functools

import jax
import jax.numpy as jnp
from jax import lax
from jax.experimental import pallas as pl
from jax.experimental.pallas import tpu as pltpu
from jax.experimental.pallas import tpu_sc as plsc

B, N, C, H, E, TOPK = 2, 2048, 1024, 16, 8, 2
DFF = 4 * C
T = B * N
DH = C // H
BLK = 128
P = 5120
NBLK = P // BLK
RB = 128
NC_SC, NS_SC = 2, 16
NW = NC_SC * NS_SC
F32 = jnp.float32


def _gelu(x):
    return 0.5 * x * (1.0 + lax.erf(x * 0.7071067811865476))


def _ln(x, g, b, eps=1e-5):
    m = jnp.mean(x, axis=-1, keepdims=True)
    v = jnp.mean((x - m) ** 2, axis=-1, keepdims=True)
    return (x - m) / jnp.sqrt(v + eps) * g + b


def _mm(a, b_mat, contract_b=1, prec=None):
    return lax.dot_general(a, b_mat, (((1,), (contract_b,)), ((), ())),
                           preferred_element_type=F32, precision=prec)


def _row_stats(t):
    m = jnp.mean(t, axis=-1, keepdims=True)
    v = jnp.mean((t - m) ** 2, axis=-1, keepdims=True)
    return m.reshape(T, 1), v.reshape(T, 1)


def _qkv_body(x_ref, m_ref, v_ref, g_ref, b_ref, w_ref, bias_ref, out_ref):
    h1 = ((x_ref[...] - m_ref[...]) / jnp.sqrt(v_ref[...] + 1e-5)
          * g_ref[...] + b_ref[...])
    out_ref[...] = _mm(h1, w_ref[...]) + bias_ref[...]


def _qkv_call(xf, m1, v1, g, b, w_qkv, b_qkv):
    return pl.pallas_call(
        _qkv_body,
        grid=(T // RB,),
        in_specs=[
            pl.BlockSpec((RB, C), lambda i: (i, 0)),
            pl.BlockSpec((RB, 1), lambda i: (i, 0)),
            pl.BlockSpec((RB, 1), lambda i: (i, 0)),
            pl.BlockSpec((1, C), lambda i: (0, 0)),
            pl.BlockSpec((1, C), lambda i: (0, 0)),
            pl.BlockSpec((3 * C, C), lambda i: (0, 0)),
            pl.BlockSpec((1, 3 * C), lambda i: (0, 0)),
        ],
        out_specs=pl.BlockSpec((RB, 3 * C), lambda i: (i, 0)),
        out_shape=jax.ShapeDtypeStruct((T, 3 * C), F32),
        compiler_params=pltpu.CompilerParams(
            dimension_semantics=("arbitrary",)),
    )(xf, m1, v1, g, b, w_qkv, b_qkv)


KC = 1024


def _attn_body(q_ref, kt_ref, v_ref, o_ref):
    q = q_ref[0]
    s0 = _mm(q, kt_ref[0][:, 0:KC], contract_b=0) * (1.0 / (DH ** 0.5))
    m0 = jnp.max(s0, axis=-1, keepdims=True)
    p0 = jnp.exp(s0 - m0)
    l0 = jnp.sum(p0, axis=-1, keepdims=True)
    o0 = _mm(p0, v_ref[0][0:KC, :], contract_b=0) * (1.0 / l0)
    for c in range(1, N // KC):
        s1 = _mm(q, kt_ref[0][:, c * KC:(c + 1) * KC],
                 contract_b=0) * (1.0 / (DH ** 0.5))
        mc = jnp.max(s1, axis=-1, keepdims=True)
        mn = jnp.maximum(m0, mc)
        ed = jnp.exp(jnp.where(m0 == mn, 0.0, m0 - mn))
        p1 = jnp.exp(s1 - mn)
        sp = jnp.sum(p1, axis=-1, keepdims=True)
        l1 = ed * l0 + sp
        acc = (ed * l0) * o0
        o0 = (_mm(p1, v_ref[0][c * KC:(c + 1) * KC, :], contract_b=0)
              + acc) * (1.0 / l1)
        m0, l0 = mn, l1
    o_ref[0] = o0


def _attn_call(q3, kt3, v3):
    QB = 256
    return pl.pallas_call(
        _attn_body,
        grid=(B * H, N // QB),
        in_specs=[
            pl.BlockSpec((1, QB, DH), lambda h, i: (h, i, 0)),
            pl.BlockSpec((1, DH, N), lambda h, i: (h, 0, 0)),
            pl.BlockSpec((1, N, DH), lambda h, i: (h, 0, 0)),
        ],
        out_specs=pl.BlockSpec((1, QB, DH), lambda h, i: (h, i, 0)),
        out_shape=jax.ShapeDtypeStruct((B * H, N, DH), F32),
        compiler_params=pltpu.CompilerParams(
            dimension_semantics=("arbitrary", "arbitrary")),
    )(q3, kt3, v3)


def _oproj_body(ao_ref, x_ref, wo_ref, bo_ref, x1_ref):
    x1_ref[...] = x_ref[...] + _mm(ao_ref[...], wo_ref[...]) + bo_ref[...]


def _oproj_call(ao, xf, w_o, b_o):
    return pl.pallas_call(
        _oproj_body,
        grid=(T // RB,),
        in_specs=[
            pl.BlockSpec((RB, C), lambda i: (i, 0)),
            pl.BlockSpec((RB, C), lambda i: (i, 0)),
            pl.BlockSpec((C, C), lambda i: (0, 0)),
            pl.BlockSpec((1, C), lambda i: (0, 0)),
        ],
        out_specs=pl.BlockSpec((RB, C), lambda i: (i, 0)),
        out_shape=jax.ShapeDtypeStruct((T, C), F32),
        compiler_params=pltpu.CompilerParams(
            dimension_semantics=("arbitrary",)),
    )(ao, xf, w_o, b_o)


def _gate_body(x1_ref, m_ref, v_ref, g_ref, b_ref, wg_ref, bg_ref,
               h2_ref, lg_ref):
    h2 = ((x1_ref[...] - m_ref[...]) / jnp.sqrt(v_ref[...] + 1e-5)
          * g_ref[...] + b_ref[...])
    h2_ref[...] = h2
    lg_ref[...] = _mm(h2, wg_ref[...]) + bg_ref[...]


def _gate_call(x1, m2, v2, g, b, wg_pad, bg_pad):
    return pl.pallas_call(
        _gate_body,
        grid=(T // RB,),
        in_specs=[
            pl.BlockSpec((RB, C), lambda i: (i, 0)),
            pl.BlockSpec((RB, 1), lambda i: (i, 0)),
            pl.BlockSpec((RB, 1), lambda i: (i, 0)),
            pl.BlockSpec((1, C), lambda i: (0, 0)),
            pl.BlockSpec((1, C), lambda i: (0, 0)),
            pl.BlockSpec((128, C), lambda i: (0, 0)),
            pl.BlockSpec((1, 128), lambda i: (0, 0)),
        ],
        out_specs=[
            pl.BlockSpec((RB, C), lambda i: (i, 0)),
            pl.BlockSpec((RB, 128), lambda i: (i, 0)),
        ],
        out_shape=[
            jax.ShapeDtypeStruct((T, C), F32),
            jax.ShapeDtypeStruct((T, 128), F32),
        ],
        compiler_params=pltpu.CompilerParams(
            dimension_semantics=("arbitrary",)),
    )(x1, m2, v2, g, b, wg_pad, bg_pad)


def _route_body(lg_ref, p_ref, bb_ref):
    CH = 256
    lg = lg_ref[...]
    lane = lax.broadcasted_iota(jnp.int32, (T, 128), 1)
    neg = jnp.float32(-1e30)
    lgm = jnp.where(lane < E, lg, neg)
    m1 = jnp.max(lgm, axis=-1, keepdims=True)
    i1 = jnp.min(jnp.where(lgm == m1, lane, 127), axis=-1, keepdims=True)
    lg2 = jnp.where(lane == i1, neg, lgm)
    m2 = jnp.max(lg2, axis=-1, keepdims=True)
    i2 = jnp.min(jnp.where(lg2 == m2, lane, 127), axis=-1, keepdims=True)
    e = jnp.maximum(i1, i2)
    onehot = (lane == e).astype(F32)

    counts = jnp.sum(onehot, axis=0, keepdims=True)
    pb = (counts.astype(jnp.int32) + (BLK - 1)) // BLK
    r0 = lax.broadcasted_iota(jnp.int32, (128, 128), 0)
    c0 = lax.broadcasted_iota(jnp.int32, (128, 128), 1)
    su = (r0 < c0).astype(F32)
    bb = _mm(pb.astype(F32), su, contract_b=0)
    bb_ref[...] = bb.astype(jnp.int32)
    base = bb * float(BLK)

    rr = lax.broadcasted_iota(jnp.int32, (CH, CH), 0)
    cc = lax.broadcasted_iota(jnp.int32, (CH, CH), 1)
    tril = (rr > cc).astype(F32)
    run = jnp.zeros((1, 128), F32)
    for c in range(T // CH):
        oh = onehot[c * CH:(c + 1) * CH, :]
        rank = _mm(tril, oh, contract_b=0)
        pos = base + run + rank
        pv = jnp.sum(oh * pos, axis=-1, keepdims=True)
        p_ref[c * CH:(c + 1) * CH, :] = pv.astype(jnp.int32)
        run = run + jnp.sum(oh, axis=0, keepdims=True)


def _route_call(lg):
    return pl.pallas_call(
        _route_body,
        out_shape=[
            jax.ShapeDtypeStruct((T, 1), jnp.int32),
            jax.ShapeDtypeStruct((1, 128), jnp.int32),
        ],
    )(lg)


_ROWS_W = T // NW
_CHUNK = 64


def _sc_wid():
    return lax.axis_index("s") * NC_SC + lax.axis_index("c")


def _sc_mesh():
    return plsc.VectorSubcoreMesh(core_axis_name="c", subcore_axis_name="s")


def _sc_dispatch(p, h2):
    @functools.partial(
        pl.kernel, mesh=_sc_mesh(),
        out_type=jax.ShapeDtypeStruct((P, C), F32),
        scratch_types=[
            pltpu.VMEM((_CHUNK,), jnp.int32),
            pltpu.VMEM((_CHUNK, C), F32),
            pltpu.SemaphoreType.DMA,
        ],
    )
    def body(p_hbm, h2_hbm, xs_hbm, pv, rv, sem):
        base = _sc_wid() * _ROWS_W
        for c in range(_ROWS_W // _CHUNK):
            off = base + c * _CHUNK
            pltpu.sync_copy(p_hbm.at[pl.ds(off, _CHUNK)], pv)
            pltpu.sync_copy(h2_hbm.at[pl.ds(off, _CHUNK)], rv)
            pltpu.async_copy(rv, xs_hbm.at[pv], sem).wait()

    return body(p, h2)


def _sc_combine(p, ys):
    @functools.partial(
        pl.kernel, mesh=_sc_mesh(),
        out_type=jax.ShapeDtypeStruct((T, C), F32),
        scratch_types=[
            pltpu.VMEM((_CHUNK,), jnp.int32),
            pltpu.VMEM((_CHUNK, C), F32),
            pltpu.SemaphoreType.DMA,
        ],
    )
    def body(p_hbm, ys_hbm, mo_hbm, pv, rv, sem):
        base = _sc_wid() * _ROWS_W
        for c in range(_ROWS_W // _CHUNK):
            off = base + c * _CHUNK
            pltpu.sync_copy(p_hbm.at[pl.ds(off, _CHUNK)], pv)
            pltpu.async_copy(ys_hbm.at[pv], rv, sem).wait()
            pltpu.sync_copy(rv, mo_hbm.at[pl.ds(off, _CHUNK)])

    return body(p, ys)


DHALF = DFF // 2


def _ffn_body(bb_ref, xs_ref, w1_ref, b1_ref, w2_ref, b2_ref, out_ref,
              acc_ref):
    d = pl.program_id(0)
    i = pl.program_id(1)
    h = _gelu(_mm(xs_ref[...], w1_ref[0]) + b1_ref[0])
    part = _mm(h, w2_ref[0])
    rows = pl.ds(i * BLK, BLK)

    @pl.when(d == 0)
    def _():
        acc_ref[rows, :] = part

    @pl.when(d == 1)
    def _():
        out_ref[...] = acc_ref[rows, :] + part + b2_ref[0]


def _expert_of(i, bb_ref):
    be = jnp.int32(0)
    for e in range(1, E):
        be = be + (i >= bb_ref[e]).astype(jnp.int32)
    return be


def _ffn_call(bb8, xs, w_e1, b_e1, w_e2, b_e2):
    grid_spec = pltpu.PrefetchScalarGridSpec(
        num_scalar_prefetch=1,
        grid=(2, NBLK),
        in_specs=[
            pl.BlockSpec((BLK, C), lambda d, i, bb: (i, 0)),
            pl.BlockSpec((1, DHALF, C),
                         lambda d, i, bb: (_expert_of(i, bb), d, 0)),
            pl.BlockSpec((1, 1, DHALF),
                         lambda d, i, bb: (_expert_of(i, bb), 0, d)),
            pl.BlockSpec((1, C, DHALF),
                         lambda d, i, bb: (_expert_of(i, bb), 0, d)),
            pl.BlockSpec((1, 1, C),
                         lambda d, i, bb: (_expert_of(i, bb), 0, 0)),
        ],
        out_specs=pl.BlockSpec((BLK, C), lambda d, i, bb: (i, 0)),
        scratch_shapes=[pltpu.VMEM((P, C), F32)],
    )
    return pl.pallas_call(
        _ffn_body,
        grid_spec=grid_spec,
        out_shape=jax.ShapeDtypeStruct((P, C), F32),
        compiler_params=pltpu.CompilerParams(
            dimension_semantics=("arbitrary", "arbitrary"),
            vmem_limit_bytes=100 * 1024 * 1024),
    )(bb8, xs, w_e1, b_e1, w_e2, b_e2)


def _mlp_body(x1_ref, mo_ref, g_ref, b_ref, w1_ref, b1_ref, w2_ref, b2_ref,
              out_ref, acc_ref):
    d = pl.program_id(0)
    i = pl.program_id(1)
    x2 = x1_ref[...] + float(TOPK) * mo_ref[...]
    h3 = _ln(x2, g_ref[...], b_ref[...])
    m = _gelu(_mm(h3, w1_ref[...]) + b1_ref[...])
    part = _mm(m, w2_ref[...])
    rows = pl.ds(i * RB, RB)

    @pl.when(d == 0)
    def _():
        acc_ref[rows, :] = part

    @pl.when(d == 1)
    def _():
        out_ref[...] = x2 + acc_ref[rows, :] + part + b2_ref[...]


def _mlp_call(x1, mo, g, b, w_m1, b_m1, w_m2, b_m2):
    return pl.pallas_call(
        _mlp_body,
        grid=(2, T // RB),
        in_specs=[
            pl.BlockSpec((RB, C), lambda d, i: (i, 0)),
            pl.BlockSpec((RB, C), lambda d, i: (i, 0)),
            pl.BlockSpec((1, C), lambda d, i: (0, 0)),
            pl.BlockSpec((1, C), lambda d, i: (0, 0)),
            pl.BlockSpec((DHALF, C), lambda d, i: (d, 0)),
            pl.BlockSpec((1, DHALF), lambda d, i: (0, d)),
            pl.BlockSpec((C, DHALF), lambda d, i: (0, d)),
            pl.BlockSpec((1, C), lambda d, i: (0, 0)),
        ],
        out_specs=pl.BlockSpec((RB, C), lambda d, i: (i, 0)),
        out_shape=jax.ShapeDtypeStruct((T, C), F32),
        scratch_shapes=[pltpu.VMEM((T, C), F32)],
        compiler_params=pltpu.CompilerParams(
            dimension_semantics=("arbitrary", "arbitrary"),
            vmem_limit_bytes=100 * 1024 * 1024),
    )(x1, mo, g, b, w_m1, b_m1, w_m2, b_m2)


def kernel(x, ln1_g, ln1_b, ln2_g, ln2_b, ln3_g, ln3_b, w_qkv, b_qkv, w_o,
           b_o, w_gate, b_gate, w_e1, b_e1, w_e2, b_e2, w_m1, b_m1, w_m2,
           b_m2):
    xf = x.reshape(T, C)
    r2 = lambda v: v.reshape(1, -1)

    m1, v1 = _row_stats(x)
    qkv = _qkv_call(xf, m1, v1, r2(ln1_g), r2(ln1_b), w_qkv, r2(b_qkv))
    q, k, v = jnp.split(qkv, 3, axis=-1)
    hs = lambda t: t.reshape(B, N, H, DH).transpose(0, 2, 1, 3).reshape(
        B * H, N, DH)
    q3, v3 = hs(q), hs(v)
    kt3 = hs(k).transpose(0, 2, 1)
    ao = _attn_call(q3, kt3, v3)
    ao = ao.reshape(B, H, N, DH).transpose(0, 2, 1, 3).reshape(T, C)

    wg_pad = jnp.zeros((128, C), F32).at[:E].set(w_gate)
    bg_pad = jnp.pad(b_gate, (0, 128 - E)).reshape(1, 128)
    x1 = _oproj_call(ao, xf, w_o, r2(b_o))
    m2, v2 = _row_stats(x1.reshape(B, N, C))
    h2, lg = _gate_call(x1, m2, v2, r2(ln2_g), r2(ln2_b), wg_pad, bg_pad)

    p2, bb = _route_call(lg)
    p = p2.reshape(T)
    bb8 = bb.reshape(128)[:E]

    xs = _sc_dispatch(p, h2)
    ys = _ffn_call(bb8, xs, w_e1, b_e1.reshape(E, 1, DFF), w_e2,
                   b_e2.reshape(E, 1, C))
    mo = _sc_combine(p, ys)

    out = _mlp_call(x1, mo, r2(ln3_g), r2(ln3_b), w_m1, r2(b_m1), w_m2,
                    r2(b_m2))
    return out.reshape(B, N, C)


# speedup vs baseline: 2.7418x; 1.2706x over previous
import functools

import jax
import jax.numpy as jnp
from jax import lax
from jax.experimental import pallas as pl
from jax.experimental.pallas import tpu as pltpu
from jax.experimental.pallas import tpu_sc as plsc

B, N, C, H, E, TOPK = 2, 2048, 1024, 16, 8, 2
DFF = 4 * C
T = B * N
DH = C // H
BLK = 256
P = T + E * BLK
NBLK = P // BLK
RB = 256
NC_SC, NS_SC = 2, 16
NW = NC_SC * NS_SC
F32 = jnp.float32


def _gelu(x):
    return 0.5 * x * (1.0 + lax.erf(x * 0.7071067811865476))


def _ln(x, g, b, eps=1e-5):
    m = jnp.mean(x, axis=-1, keepdims=True)
    v = jnp.mean((x - m) ** 2, axis=-1, keepdims=True)
    return (x - m) / jnp.sqrt(v + eps) * g + b


def _mm(a, b_mat, contract_b=1, prec=None):
    return lax.dot_general(a, b_mat, (((1,), (contract_b,)), ((), ())),
                           preferred_element_type=F32, precision=prec)


def _row_stats(t):
    m = jnp.mean(t, axis=-1, keepdims=True)
    v = jnp.mean((t - m) ** 2, axis=-1, keepdims=True)
    return m.reshape(T, 1), v.reshape(T, 1)


def _qkv_body(x_ref, m_ref, v_ref, g_ref, b_ref, w_ref, bias_ref, out_ref):
    h1 = ((x_ref[...] - m_ref[...]) / jnp.sqrt(v_ref[...] + 1e-5)
          * g_ref[...] + b_ref[...])
    out_ref[...] = _mm(h1, w_ref[...]) + bias_ref[...]


def _qkv_call(xf, m1, v1, g, b, w_qkv, b_qkv):
    return pl.pallas_call(
        _qkv_body,
        grid=(T // RB,),
        in_specs=[
            pl.BlockSpec((RB, C), lambda i: (i, 0)),
            pl.BlockSpec((RB, 1), lambda i: (i, 0)),
            pl.BlockSpec((RB, 1), lambda i: (i, 0)),
            pl.BlockSpec((1, C), lambda i: (0, 0)),
            pl.BlockSpec((1, C), lambda i: (0, 0)),
            pl.BlockSpec((3 * C, C), lambda i: (0, 0)),
            pl.BlockSpec((1, 3 * C), lambda i: (0, 0)),
        ],
        out_specs=pl.BlockSpec((RB, 3 * C), lambda i: (i, 0)),
        out_shape=jax.ShapeDtypeStruct((T, 3 * C), F32),
        compiler_params=pltpu.CompilerParams(
            dimension_semantics=("arbitrary",)),
    )(xf, m1, v1, g, b, w_qkv, b_qkv)


KC = 1024


def _attn_body(q_ref, kt_ref, v_ref, o_ref):
    q = q_ref[0]
    s0 = _mm(q, kt_ref[0][:, 0:KC], contract_b=0) * (1.0 / (DH ** 0.5))
    m0 = jnp.max(s0, axis=-1, keepdims=True)
    p0 = jnp.exp(s0 - m0)
    l0 = jnp.sum(p0, axis=-1, keepdims=True)
    o0 = _mm(p0, v_ref[0][0:KC, :], contract_b=0) * (1.0 / l0)
    for c in range(1, N // KC):
        s1 = _mm(q, kt_ref[0][:, c * KC:(c + 1) * KC],
                 contract_b=0) * (1.0 / (DH ** 0.5))
        mc = jnp.max(s1, axis=-1, keepdims=True)
        mn = jnp.maximum(m0, mc)
        ed = jnp.exp(jnp.where(m0 == mn, 0.0, m0 - mn))
        p1 = jnp.exp(s1 - mn)
        sp = jnp.sum(p1, axis=-1, keepdims=True)
        l1 = ed * l0 + sp
        acc = (ed * l0) * o0
        o0 = (_mm(p1, v_ref[0][c * KC:(c + 1) * KC, :], contract_b=0)
              + acc) * (1.0 / l1)
        m0, l0 = mn, l1
    o_ref[0] = o0


def _attn_call(q3, kt3, v3):
    QB = 256
    return pl.pallas_call(
        _attn_body,
        grid=(B * H, N // QB),
        in_specs=[
            pl.BlockSpec((1, QB, DH), lambda h, i: (h, i, 0)),
            pl.BlockSpec((1, DH, N), lambda h, i: (h, 0, 0)),
            pl.BlockSpec((1, N, DH), lambda h, i: (h, 0, 0)),
        ],
        out_specs=pl.BlockSpec((1, QB, DH), lambda h, i: (h, i, 0)),
        out_shape=jax.ShapeDtypeStruct((B * H, N, DH), F32),
        compiler_params=pltpu.CompilerParams(
            dimension_semantics=("arbitrary", "arbitrary")),
    )(q3, kt3, v3)


def _oproj_body(ao_ref, x_ref, wo_ref, bo_ref, x1_ref):
    x1_ref[...] = x_ref[...] + _mm(ao_ref[...], wo_ref[...]) + bo_ref[...]


def _oproj_call(ao, xf, w_o, b_o):
    return pl.pallas_call(
        _oproj_body,
        grid=(T // RB,),
        in_specs=[
            pl.BlockSpec((RB, C), lambda i: (i, 0)),
            pl.BlockSpec((RB, C), lambda i: (i, 0)),
            pl.BlockSpec((C, C), lambda i: (0, 0)),
            pl.BlockSpec((1, C), lambda i: (0, 0)),
        ],
        out_specs=pl.BlockSpec((RB, C), lambda i: (i, 0)),
        out_shape=jax.ShapeDtypeStruct((T, C), F32),
        compiler_params=pltpu.CompilerParams(
            dimension_semantics=("arbitrary",)),
    )(ao, xf, w_o, b_o)


def _gate_body(x1_ref, m_ref, v_ref, g_ref, b_ref, wg_ref, bg_ref,
               h2_ref, lg_ref):
    h2 = ((x1_ref[...] - m_ref[...]) / jnp.sqrt(v_ref[...] + 1e-5)
          * g_ref[...] + b_ref[...])
    h2_ref[...] = h2
    lg_ref[...] = _mm(h2, wg_ref[...]) + bg_ref[...]


def _gate_call(x1, m2, v2, g, b, wg_pad, bg_pad):
    return pl.pallas_call(
        _gate_body,
        grid=(T // RB,),
        in_specs=[
            pl.BlockSpec((RB, C), lambda i: (i, 0)),
            pl.BlockSpec((RB, 1), lambda i: (i, 0)),
            pl.BlockSpec((RB, 1), lambda i: (i, 0)),
            pl.BlockSpec((1, C), lambda i: (0, 0)),
            pl.BlockSpec((1, C), lambda i: (0, 0)),
            pl.BlockSpec((128, C), lambda i: (0, 0)),
            pl.BlockSpec((1, 128), lambda i: (0, 0)),
        ],
        out_specs=[
            pl.BlockSpec((RB, C), lambda i: (i, 0)),
            pl.BlockSpec((RB, 128), lambda i: (i, 0)),
        ],
        out_shape=[
            jax.ShapeDtypeStruct((T, C), F32),
            jax.ShapeDtypeStruct((T, 128), F32),
        ],
        compiler_params=pltpu.CompilerParams(
            dimension_semantics=("arbitrary",)),
    )(x1, m2, v2, g, b, wg_pad, bg_pad)


def _route_body(lg_ref, p_ref, bb_ref):
    CH = 256
    lg = lg_ref[...]
    lane = lax.broadcasted_iota(jnp.int32, (T, 128), 1)
    neg = jnp.float32(-1e30)
    lgm = jnp.where(lane < E, lg, neg)
    m1 = jnp.max(lgm, axis=-1, keepdims=True)
    i1 = jnp.min(jnp.where(lgm == m1, lane, 127), axis=-1, keepdims=True)
    lg2 = jnp.where(lane == i1, neg, lgm)
    m2 = jnp.max(lg2, axis=-1, keepdims=True)
    i2 = jnp.min(jnp.where(lg2 == m2, lane, 127), axis=-1, keepdims=True)
    e = jnp.maximum(i1, i2)
    onehot = (lane == e).astype(F32)

    counts = jnp.sum(onehot, axis=0, keepdims=True)
    pb = (counts.astype(jnp.int32) + (BLK - 1)) // BLK
    r0 = lax.broadcasted_iota(jnp.int32, (128, 128), 0)
    c0 = lax.broadcasted_iota(jnp.int32, (128, 128), 1)
    su = (r0 < c0).astype(F32)
    bb = _mm(pb.astype(F32), su, contract_b=0)
    bb_ref[...] = bb.astype(jnp.int32)
    base = bb * float(BLK)

    rr = lax.broadcasted_iota(jnp.int32, (CH, CH), 0)
    cc = lax.broadcasted_iota(jnp.int32, (CH, CH), 1)
    tril = (rr > cc).astype(F32)
    run = jnp.zeros((1, 128), F32)
    for c in range(T // CH):
        oh = onehot[c * CH:(c + 1) * CH, :]
        rank = _mm(tril, oh, contract_b=0)
        pos = base + run + rank
        pv = jnp.sum(oh * pos, axis=-1, keepdims=True)
        p_ref[c * CH:(c + 1) * CH, :] = pv.astype(jnp.int32)
        run = run + jnp.sum(oh, axis=0, keepdims=True)


def _route_call(lg):
    return pl.pallas_call(
        _route_body,
        out_shape=[
            jax.ShapeDtypeStruct((T, 1), jnp.int32),
            jax.ShapeDtypeStruct((1, 128), jnp.int32),
        ],
    )(lg)


_ROWS_W = T // NW
_CHUNK = 32
_NCH = _ROWS_W // _CHUNK


def _sc_wid():
    return lax.axis_index("s") * NC_SC + lax.axis_index("c")


def _sc_mesh():
    return plsc.VectorSubcoreMesh(core_axis_name="c", subcore_axis_name="s")


_SC_SCRATCH = [
    pltpu.VMEM((_CHUNK,), jnp.int32),
    pltpu.VMEM((_CHUNK,), jnp.int32),
    pltpu.VMEM((_CHUNK, C), F32),
    pltpu.VMEM((_CHUNK, C), F32),
    pltpu.SemaphoreType.DMA,
    pltpu.SemaphoreType.DMA,
]


def _sc_dispatch(p, h2):
    @functools.partial(
        pl.kernel, mesh=_sc_mesh(),
        out_type=jax.ShapeDtypeStruct((P, C), F32),
        scratch_types=_SC_SCRATCH,
    )
    def body(p_hbm, h2_hbm, xs_hbm, pv0, pv1, rv0, rv1, sem0, sem1):
        base = _sc_wid() * _ROWS_W
        pvs, rvs, sems = (pv0, pv1), (rv0, rv1), (sem0, sem1)
        cps = [None, None]
        for c in range(_NCH):
            b = c % 2
            off = base + c * _CHUNK
            if cps[b] is not None:
                cps[b].wait()
            pltpu.sync_copy(p_hbm.at[pl.ds(off, _CHUNK)], pvs[b])
            pltpu.sync_copy(h2_hbm.at[pl.ds(off, _CHUNK)], rvs[b])
            cps[b] = pltpu.async_copy(rvs[b], xs_hbm.at[pvs[b]], sems[b])
        for cp in cps:
            cp.wait()

    return body(p, h2)


def _sc_combine(p, ys):
    @functools.partial(
        pl.kernel, mesh=_sc_mesh(),
        out_type=jax.ShapeDtypeStruct((T, C), F32),
        scratch_types=_SC_SCRATCH,
    )
    def body(p_hbm, ys_hbm, mo_hbm, pv0, pv1, rv0, rv1, sem0, sem1):
        base = _sc_wid() * _ROWS_W
        pvs, rvs, sems = (pv0, pv1), (rv0, rv1), (sem0, sem1)
        cps = [None, None]
        for c in range(_NCH):
            b = c % 2
            off = base + c * _CHUNK
            if cps[b] is not None:
                cps[b].wait()
                pltpu.sync_copy(rvs[b],
                                mo_hbm.at[pl.ds(off - 2 * _CHUNK, _CHUNK)])
            pltpu.sync_copy(p_hbm.at[pl.ds(off, _CHUNK)], pvs[b])
            cps[b] = pltpu.async_copy(ys_hbm.at[pvs[b]], rvs[b], sems[b])
        for c in range(_NCH - 2, _NCH):
            b = c % 2
            cps[b].wait()
            pltpu.sync_copy(rvs[b], mo_hbm.at[pl.ds(base + c * _CHUNK,
                                                    _CHUNK)])

    return body(p, ys)


DHALF = DFF // 2


def _ffn_body(bb_ref, xs_ref, w1_ref, b1_ref, w2_ref, b2_ref, out_ref,
              acc_ref):
    d = pl.program_id(0)
    i = pl.program_id(1)
    h = _gelu(_mm(xs_ref[...], w1_ref[0]) + b1_ref[0])
    part = _mm(h, w2_ref[0])
    rows = pl.ds(i * BLK, BLK)

    @pl.when(d == 0)
    def _():
        acc_ref[rows, :] = part

    @pl.when(d == 1)
    def _():
        out_ref[...] = acc_ref[rows, :] + part + b2_ref[0]


def _expert_of(i, bb_ref):
    be = jnp.int32(0)
    for e in range(1, E):
        be = be + (i >= bb_ref[e]).astype(jnp.int32)
    return be


def _ffn_call(bb8, xs, w_e1, b_e1, w_e2, b_e2):
    grid_spec = pltpu.PrefetchScalarGridSpec(
        num_scalar_prefetch=1,
        grid=(2, NBLK),
        in_specs=[
            pl.BlockSpec((BLK, C), lambda d, i, bb: (i, 0)),
            pl.BlockSpec((1, DHALF, C),
                         lambda d, i, bb: (_expert_of(i, bb), d, 0)),
            pl.BlockSpec((1, 1, DHALF),
                         lambda d, i, bb: (_expert_of(i, bb), 0, d)),
            pl.BlockSpec((1, C, DHALF),
                         lambda d, i, bb: (_expert_of(i, bb), 0, d)),
            pl.BlockSpec((1, 1, C),
                         lambda d, i, bb: (_expert_of(i, bb), 0, 0)),
        ],
        out_specs=pl.BlockSpec((BLK, C), lambda d, i, bb: (i, 0)),
        scratch_shapes=[pltpu.VMEM((P, C), F32)],
    )
    return pl.pallas_call(
        _ffn_body,
        grid_spec=grid_spec,
        out_shape=jax.ShapeDtypeStruct((P, C), F32),
        compiler_params=pltpu.CompilerParams(
            dimension_semantics=("arbitrary", "arbitrary"),
            vmem_limit_bytes=100 * 1024 * 1024),
    )(bb8, xs, w_e1, b_e1, w_e2, b_e2)


def _mlp_body(x1_ref, mo_ref, g_ref, b_ref, w1_ref, b1_ref, w2_ref, b2_ref,
              out_ref, acc_ref):
    d = pl.program_id(0)
    i = pl.program_id(1)
    x2 = x1_ref[...] + float(TOPK) * mo_ref[...]
    h3 = _ln(x2, g_ref[...], b_ref[...])
    m = _gelu(_mm(h3, w1_ref[...]) + b1_ref[...])
    part = _mm(m, w2_ref[...])
    rows = pl.ds(i * RB, RB)

    @pl.when(d == 0)
    def _():
        acc_ref[rows, :] = part

    @pl.when(d == 1)
    def _():
        out_ref[...] = x2 + acc_ref[rows, :] + part + b2_ref[...]


def _mlp_call(x1, mo, g, b, w_m1, b_m1, w_m2, b_m2):
    return pl.pallas_call(
        _mlp_body,
        grid=(2, T // RB),
        in_specs=[
            pl.BlockSpec((RB, C), lambda d, i: (i, 0)),
            pl.BlockSpec((RB, C), lambda d, i: (i, 0)),
            pl.BlockSpec((1, C), lambda d, i: (0, 0)),
            pl.BlockSpec((1, C), lambda d, i: (0, 0)),
            pl.BlockSpec((DHALF, C), lambda d, i: (d, 0)),
            pl.BlockSpec((1, DHALF), lambda d, i: (0, d)),
            pl.BlockSpec((C, DHALF), lambda d, i: (0, d)),
            pl.BlockSpec((1, C), lambda d, i: (0, 0)),
        ],
        out_specs=pl.BlockSpec((RB, C), lambda d, i: (i, 0)),
        out_shape=jax.ShapeDtypeStruct((T, C), F32),
        scratch_shapes=[pltpu.VMEM((T, C), F32)],
        compiler_params=pltpu.CompilerParams(
            dimension_semantics=("arbitrary", "arbitrary"),
            vmem_limit_bytes=100 * 1024 * 1024),
    )(x1, mo, g, b, w_m1, b_m1, w_m2, b_m2)


def kernel(x, ln1_g, ln1_b, ln2_g, ln2_b, ln3_g, ln3_b, w_qkv, b_qkv, w_o,
           b_o, w_gate, b_gate, w_e1, b_e1, w_e2, b_e2, w_m1, b_m1, w_m2,
           b_m2):
    xf = x.reshape(T, C)
    r2 = lambda v: v.reshape(1, -1)

    m1, v1 = _row_stats(x)
    qkv = _qkv_call(xf, m1, v1, r2(ln1_g), r2(ln1_b), w_qkv, r2(b_qkv))
    q, k, v = jnp.split(qkv, 3, axis=-1)
    hs = lambda t: t.reshape(B, N, H, DH).transpose(0, 2, 1, 3).reshape(
        B * H, N, DH)
    q3, v3 = hs(q), hs(v)
    kt3 = hs(k).transpose(0, 2, 1)
    ao = _attn_call(q3, kt3, v3)
    ao = ao.reshape(B, H, N, DH).transpose(0, 2, 1, 3).reshape(T, C)

    wg_pad = jnp.zeros((128, C), F32).at[:E].set(w_gate)
    bg_pad = jnp.pad(b_gate, (0, 128 - E)).reshape(1, 128)
    x1 = _oproj_call(ao, xf, w_o, r2(b_o))
    m2, v2 = _row_stats(x1.reshape(B, N, C))
    h2, lg = _gate_call(x1, m2, v2, r2(ln2_g), r2(ln2_b), wg_pad, bg_pad)

    p2, bb = _route_call(lg)
    p = p2.reshape(T)
    bb8 = bb.reshape(128)[:E]

    xs = _sc_dispatch(p, h2)
    ys = _ffn_call(bb8, xs, w_e1, b_e1.reshape(E, 1, DFF), w_e2,
                   b_e2.reshape(E, 1, C))
    mo = _sc_combine(p, ys)

    out = _mlp_call(x1, mo, r2(ln3_g), r2(ln3_b), w_m1, r2(b_m1), w_m2,
                    r2(b_m2))
    return out.reshape(B, N, C)
